# baseline (speedup 1.0000x reference)
import functools
import jax
import jax.numpy as jnp
from jax import lax
from jax.experimental import pallas as pl
from jax.experimental.pallas import tpu as pltpu
from jax.experimental.pallas import tpu_sc as plsc

B = 128
N = 8192
NW = 32
RPW = B // NW
L = 16
UNROLL = 16
VPR = N // L


def _sc_body(x_hbm, out_hbm, ibuf, obuf, sem):
    wid = lax.axis_index("s") * 2 + lax.axis_index("c")
    base = wid * RPW
    pltpu.async_copy(x_hbm.at[pl.ds(base, RPW), :], ibuf, sem).wait()
    for r in range(RPW):
        def maxstep(i, m):
            vs = [ibuf[r, pl.ds((i * UNROLL + u) * L, L)] for u in range(UNROLL)]
            for v in vs:
                m = jnp.maximum(m, v)
            return m
        m16 = lax.fori_loop(0, VPR // UNROLL,
                            maxstep, jnp.full((L,), -jnp.inf, jnp.float32))
        mxv = m16
        lanes = lax.iota(jnp.int32, L)
        for s in (8, 4, 2, 1):
            perm = lax.gather(
                mxv, (lanes ^ s)[:, None],
                lax.GatherDimensionNumbers(
                    offset_dims=(), collapsed_slice_dims=(0,),
                    start_index_map=(0,)),
                slice_sizes=(1,),
                mode=lax.GatherScatterMode.PROMISE_IN_BOUNDS)
            mxv = jnp.maximum(mxv, perm)
        def onestep(i, carry):
            for u in range(UNROLL):
                sl = pl.ds((i * UNROLL + u) * L, L)
                v = ibuf[r, sl]
                obuf[r, sl] = jnp.where(v == mxv, 1.0, 0.0).astype(jnp.float32)
            return carry
        lax.fori_loop(0, VPR // UNROLL, onestep, jnp.int32(0))
    pltpu.async_copy(obuf, out_hbm.at[pl.ds(base, RPW), :], sem).wait()


def kernel(attack_logits):
    k = functools.partial(
        pl.kernel,
        out_type=jax.ShapeDtypeStruct((B, N), jnp.float32),
        mesh=plsc.VectorSubcoreMesh(core_axis_name="c", subcore_axis_name="s"),
        scratch_types=[
            pltpu.VMEM((RPW, N), jnp.float32),
            pltpu.VMEM((RPW, N), jnp.float32),
            pltpu.SemaphoreType.DMA,
        ],
    )(_sc_body)
    return k(attack_logits)


# speedup vs baseline: 7.1684x; 7.1684x over previous
import jax
import jax.numpy as jnp
from jax.experimental import pallas as pl
from jax.experimental.pallas import tpu as pltpu

NC = 8
CR = 16


def _select_kernel(x_hbm, out_hbm, ibuf, obuf, in_sems, out_sems):
    for i in range(NC):
        pltpu.make_async_copy(
            x_hbm.at[pl.ds(i * CR, CR), :], ibuf.at[i], in_sems.at[i]
        ).start()
    for i in range(NC):
        pltpu.make_async_copy(
            x_hbm.at[pl.ds(i * CR, CR), :], ibuf.at[i], in_sems.at[i]
        ).wait()
        x = ibuf[i]
        mx = jnp.max(x, axis=-1, keepdims=True)
        obuf[i] = (x == mx).astype(jnp.float32)
        pltpu.make_async_copy(
            obuf.at[i], out_hbm.at[pl.ds(i * CR, CR), :], out_sems.at[i]
        ).start()
    for i in range(NC):
        pltpu.make_async_copy(
            obuf.at[i], out_hbm.at[pl.ds(i * CR, CR), :], out_sems.at[i]
        ).wait()


def kernel(attack_logits):
    b, n = attack_logits.shape
    return pl.pallas_call(
        _select_kernel,
        in_specs=[pl.BlockSpec(memory_space=pl.ANY)],
        out_specs=pl.BlockSpec(memory_space=pl.ANY),
        out_shape=jax.ShapeDtypeStruct((b, n), jnp.float32),
        scratch_shapes=[
            pltpu.VMEM((NC, CR, n), jnp.float32),
            pltpu.VMEM((NC, CR, n), jnp.float32),
            pltpu.SemaphoreType.DMA((NC,)),
            pltpu.SemaphoreType.DMA((NC,)),
        ],
    )(attack_logits)
